# baseline (speedup 1.0000x reference)
import numpy as np
import jax
import jax.numpy as jnp
from jax import lax
from jax.experimental import pallas as pl
from jax.experimental.pallas import tpu as pltpu

_B = 256


def _conv1_idx():
    idx = np.full((3456, 785), 150, np.int32)
    for c in range(6):
        for py in range(2):
            for px in range(2):
                base = (c * 4 + py * 2 + px) * 144
                for yh in range(12):
                    for xh in range(12):
                        r = base + yh * 12 + xh
                        for ki in range(5):
                            for kj in range(5):
                                col = (2 * yh + py + ki) * 28 + (2 * xh + px + kj)
                                idx[r, col] = c * 25 + ki * 5 + kj
    idx[:, 784] = 151 + np.repeat(np.arange(6), 576)
    return idx


def _conv2_idx():
    idx = np.full((768, 865), 1800, np.int32)
    for co in range(12):
        for qy in range(2):
            for qx in range(2):
                base = (co * 4 + qy * 2 + qx) * 16
                for yq in range(4):
                    for xq in range(4):
                        r = base + yq * 4 + xq
                        for cin in range(6):
                            for ki in range(5):
                                for kj in range(5):
                                    col = (cin * 144
                                           + (2 * yq + qy + ki) * 12
                                           + (2 * xq + qx + kj))
                                    idx[r, col] = co * 150 + cin * 25 + ki * 5 + kj
    idx[:, 864] = 1801 + np.repeat(np.arange(12), 64)
    return idx


_IDX1 = _conv1_idx()
_IDX2 = _conv2_idx()


def _lenet_body(xb_ref, w1a_ref, w2a_ref, w1f_ref, w2f_ref, w3f_ref, out_ref):
    f32 = jnp.float32
    bf16 = jnp.bfloat16
    ones_col = jnp.ones((_B, 1), f32)
    ones_row = jnp.ones((1, _B), f32)

    xaug = jnp.concatenate([xb_ref[...], ones_col], axis=1).astype(bf16)
    h1 = lax.dot_general(w1a_ref[...], xaug, (((1,), (1,)), ((), ())),
                         preferred_element_type=f32)
    h1 = jnp.maximum(h1, 0.0)

    p1 = jnp.concatenate(
        [jnp.maximum(
            jnp.maximum(h1[(c * 4 + 0) * 144:(c * 4 + 1) * 144],
                        h1[(c * 4 + 1) * 144:(c * 4 + 2) * 144]),
            jnp.maximum(h1[(c * 4 + 2) * 144:(c * 4 + 3) * 144],
                        h1[(c * 4 + 3) * 144:(c * 4 + 4) * 144]))
         for c in range(6)] + [ones_row], axis=0).astype(bf16)

    h2 = jnp.dot(w2a_ref[...], p1, preferred_element_type=f32)
    h2 = jnp.maximum(h2, 0.0)

    p2 = jnp.concatenate(
        [jnp.maximum(
            jnp.maximum(h2[(c * 4 + 0) * 16:(c * 4 + 1) * 16],
                        h2[(c * 4 + 1) * 16:(c * 4 + 2) * 16]),
            jnp.maximum(h2[(c * 4 + 2) * 16:(c * 4 + 3) * 16],
                        h2[(c * 4 + 3) * 16:(c * 4 + 4) * 16]))
         for c in range(12)] + [ones_row], axis=0)

    h3 = jnp.maximum(jnp.dot(w1f_ref[...], p2, preferred_element_type=f32), 0.0)
    h3 = jnp.concatenate([h3, ones_row], axis=0)
    h4 = jnp.maximum(jnp.dot(w2f_ref[...], h3, preferred_element_type=f32), 0.0)
    h4 = jnp.concatenate([h4, ones_row], axis=0)
    out_ref[0] = jnp.dot(w3f_ref[...], h4, preferred_element_type=f32)


def kernel(x, conv1_w, conv1_b, conv2_w, conv2_b,
           fc1_w, fc1_b, fc2_w, fc2_b, out_w, out_b):
    f32 = jnp.float32
    bf16 = jnp.bfloat16
    n = x.shape[0]
    nt = -(-n // _B)
    npad = nt * _B

    x2 = x.astype(f32).reshape(n, 784)
    if npad != n:
        x2 = jnp.pad(x2, ((0, npad - n), (0, 0)))

    v1 = jnp.concatenate([conv1_w.astype(f32).reshape(150),
                          jnp.zeros((1,), f32), conv1_b.astype(f32)]).astype(bf16)
    w1a = v1[_IDX1]
    v2 = jnp.concatenate([conv2_w.astype(f32).reshape(1800),
                          jnp.zeros((1,), f32), conv2_b.astype(f32)]).astype(bf16)
    w2a = v2[_IDX2]

    w1f = jnp.concatenate([fc1_w.astype(f32), fc1_b.astype(f32)[:, None]], axis=1)
    w2f = jnp.concatenate([fc2_w.astype(f32), fc2_b.astype(f32)[:, None]], axis=1)
    w3f = jnp.concatenate([out_w.astype(f32), out_b.astype(f32)[:, None]], axis=1)

    def whole(a):
        return pl.BlockSpec(a.shape, lambda i: (0,) * a.ndim)

    out = pl.pallas_call(
        _lenet_body,
        out_shape=jax.ShapeDtypeStruct((nt, 10, _B), f32),
        grid=(nt,),
        in_specs=[
            pl.BlockSpec((_B, 784), lambda i: (i, 0)),
            whole(w1a), whole(w2a), whole(w1f), whole(w2f), whole(w3f),
        ],
        out_specs=pl.BlockSpec((1, 10, _B), lambda i: (i, 0, 0)),
        compiler_params=pltpu.CompilerParams(
            dimension_semantics=("parallel",),
            vmem_limit_bytes=48 * 1024 * 1024,
        ),
    )(x2, w1a, w2a, w1f, w2f, w3f)

    return jnp.transpose(out, (0, 2, 1)).reshape(npad, 10)[:n]


# speedup vs baseline: 47.5443x; 47.5443x over previous
import numpy as np
import jax
import jax.numpy as jnp
from jax import lax
from jax.experimental import pallas as pl
from jax.experimental.pallas import tpu as pltpu

_B = 256


def _band(nout, nin):
    b = np.zeros((2, 5, nout, nin), np.float32)
    for p in range(2):
        for k in range(5):
            for y in range(nout):
                b[p, k, y, 2 * y + p + k] = 1.0
    return b


_BAND1 = _band(12, 28)
_BAND2 = _band(4, 12)


def _lenet_body(xb_ref, w1a_ref, w2a_ref, w1f_ref, w2f_ref, w3f_ref, out_ref):
    f32 = jnp.float32
    bf16 = jnp.bfloat16
    ones_col = jnp.ones((_B, 1), f32)
    ones_row = jnp.ones((1, _B), f32)

    xaug = jnp.concatenate([xb_ref[...], ones_col], axis=1).astype(bf16)
    h1 = lax.dot_general(w1a_ref[...], xaug, (((1,), (1,)), ((), ())),
                         preferred_element_type=f32)
    h1 = jnp.maximum(h1, 0.0)

    p1 = jnp.concatenate(
        [jnp.maximum(
            jnp.maximum(h1[(c * 4 + 0) * 144:(c * 4 + 1) * 144],
                        h1[(c * 4 + 1) * 144:(c * 4 + 2) * 144]),
            jnp.maximum(h1[(c * 4 + 2) * 144:(c * 4 + 3) * 144],
                        h1[(c * 4 + 3) * 144:(c * 4 + 4) * 144]))
         for c in range(6)] + [ones_row], axis=0).astype(bf16)

    h2 = jnp.dot(w2a_ref[...], p1, preferred_element_type=f32)
    h2 = jnp.maximum(h2, 0.0)

    p2 = jnp.concatenate(
        [jnp.maximum(
            jnp.maximum(h2[(c * 4 + 0) * 16:(c * 4 + 1) * 16],
                        h2[(c * 4 + 1) * 16:(c * 4 + 2) * 16]),
            jnp.maximum(h2[(c * 4 + 2) * 16:(c * 4 + 3) * 16],
                        h2[(c * 4 + 3) * 16:(c * 4 + 4) * 16]))
         for c in range(12)] + [ones_row], axis=0)

    h3 = jnp.maximum(jnp.dot(w1f_ref[...], p2, preferred_element_type=f32), 0.0)
    h3 = jnp.concatenate([h3, ones_row], axis=0)
    h4 = jnp.maximum(jnp.dot(w2f_ref[...], h3, preferred_element_type=f32), 0.0)
    h4 = jnp.concatenate([h4, ones_row], axis=0)
    out_ref[0] = jnp.dot(w3f_ref[...], h4, preferred_element_type=f32)


def kernel(x, conv1_w, conv1_b, conv2_w, conv2_b,
           fc1_w, fc1_b, fc2_w, fc2_b, out_w, out_b):
    f32 = jnp.float32
    bf16 = jnp.bfloat16
    n = x.shape[0]
    nt = -(-n // _B)
    npad = nt * _B

    x2 = x.astype(f32).reshape(n, 784)
    if npad != n:
        x2 = jnp.pad(x2, ((0, npad - n), (0, 0)))

    b1a, b1b = jnp.asarray(_BAND1), jnp.asarray(_BAND1)
    w1core = jnp.einsum('ckl,pkyi,qlxj->cpqyxij',
                        conv1_w.astype(f32).reshape(6, 5, 5), b1a, b1b,
                        ).reshape(3456, 784)
    w1a = jnp.concatenate(
        [w1core, jnp.repeat(conv1_b.astype(f32), 576)[:, None]],
        axis=1).astype(bf16)
    b2a, b2b = jnp.asarray(_BAND2), jnp.asarray(_BAND2)
    w2core = jnp.einsum('ockl,pkyi,qlxj->opqyxcij',
                        conv2_w.astype(f32), b2a, b2b).reshape(768, 864)
    w2a = jnp.concatenate(
        [w2core, jnp.repeat(conv2_b.astype(f32), 64)[:, None]],
        axis=1).astype(bf16)

    w1f = jnp.concatenate([fc1_w.astype(f32), fc1_b.astype(f32)[:, None]], axis=1)
    w2f = jnp.concatenate([fc2_w.astype(f32), fc2_b.astype(f32)[:, None]], axis=1)
    w3f = jnp.concatenate([out_w.astype(f32), out_b.astype(f32)[:, None]], axis=1)

    def whole(a):
        return pl.BlockSpec(a.shape, lambda i: (0,) * a.ndim)

    out = pl.pallas_call(
        _lenet_body,
        out_shape=jax.ShapeDtypeStruct((nt, 10, _B), f32),
        grid=(nt,),
        in_specs=[
            pl.BlockSpec((_B, 784), lambda i: (i, 0)),
            whole(w1a), whole(w2a), whole(w1f), whole(w2f), whole(w3f),
        ],
        out_specs=pl.BlockSpec((1, 10, _B), lambda i: (i, 0, 0)),
        compiler_params=pltpu.CompilerParams(
            dimension_semantics=("parallel",),
            vmem_limit_bytes=48 * 1024 * 1024,
        ),
    )(x2, w1a, w2a, w1f, w2f, w3f)

    return jnp.transpose(out, (0, 2, 1)).reshape(npad, 10)[:n]


# speedup vs baseline: 65.7444x; 1.3828x over previous
import ml_dtypes
import numpy as np
import jax
import jax.numpy as jnp
from jax import lax
from jax.experimental import pallas as pl
from jax.experimental.pallas import tpu as pltpu

_B = 256


def _band(nout, nin):
    b = np.zeros((2, 5, nout, nin), np.float32)
    for p in range(2):
        for k in range(5):
            for y in range(nout):
                b[p, k, y, 2 * y + p + k] = 1.0
    return b


_BAND1 = _band(12, 28)
_BAND2 = _band(4, 12)

_KB1 = np.einsum('pkyi,qlxj->klpqyxij', _BAND1, _BAND1).reshape(
    25, 4 * 144 * 784).astype(ml_dtypes.bfloat16)
_KB2 = np.einsum('pkyi,qlxj->klpqyxij', _BAND2, _BAND2).reshape(
    25, 4 * 16 * 144).astype(ml_dtypes.bfloat16)


def _lenet_body(xb_ref, w1a_ref, w2a_ref, w1f_ref, w2f_ref, w3f_ref, out_ref):
    f32 = jnp.float32
    bf16 = jnp.bfloat16
    ones_col = jnp.ones((_B, 1), f32)
    ones_row = jnp.ones((1, _B), f32)

    xaug = jnp.concatenate([xb_ref[...], ones_col], axis=1).astype(bf16)
    h1 = lax.dot_general(w1a_ref[...], xaug, (((1,), (1,)), ((), ())),
                         preferred_element_type=f32)
    h1 = jnp.maximum(h1, 0.0)

    p1 = jnp.concatenate(
        [jnp.maximum(
            jnp.maximum(h1[(c * 4 + 0) * 144:(c * 4 + 1) * 144],
                        h1[(c * 4 + 1) * 144:(c * 4 + 2) * 144]),
            jnp.maximum(h1[(c * 4 + 2) * 144:(c * 4 + 3) * 144],
                        h1[(c * 4 + 3) * 144:(c * 4 + 4) * 144]))
         for c in range(6)] + [ones_row], axis=0).astype(bf16)

    h2 = jnp.dot(w2a_ref[...], p1, preferred_element_type=f32)
    h2 = jnp.maximum(h2, 0.0)

    p2 = jnp.concatenate(
        [jnp.maximum(
            jnp.maximum(h2[(c * 4 + 0) * 16:(c * 4 + 1) * 16],
                        h2[(c * 4 + 1) * 16:(c * 4 + 2) * 16]),
            jnp.maximum(h2[(c * 4 + 2) * 16:(c * 4 + 3) * 16],
                        h2[(c * 4 + 3) * 16:(c * 4 + 4) * 16]))
         for c in range(12)] + [ones_row], axis=0)

    h3 = jnp.maximum(jnp.dot(w1f_ref[...], p2, preferred_element_type=f32), 0.0)
    h3 = jnp.concatenate([h3, ones_row], axis=0)
    h4 = jnp.maximum(jnp.dot(w2f_ref[...], h3, preferred_element_type=f32), 0.0)
    h4 = jnp.concatenate([h4, ones_row], axis=0)
    out_ref[0] = jnp.dot(w3f_ref[...], h4, preferred_element_type=f32)


def kernel(x, conv1_w, conv1_b, conv2_w, conv2_b,
           fc1_w, fc1_b, fc2_w, fc2_b, out_w, out_b):
    f32 = jnp.float32
    bf16 = jnp.bfloat16
    n = x.shape[0]
    nt = -(-n // _B)
    npad = nt * _B

    x2 = x.astype(f32).reshape(n, 784)
    if npad != n:
        x2 = jnp.pad(x2, ((0, npad - n), (0, 0)))

    w1core = jnp.dot(conv1_w.astype(bf16).reshape(6, 25), jnp.asarray(_KB1),
                     preferred_element_type=f32).reshape(3456, 784)
    w1a = jnp.concatenate(
        [w1core, jnp.repeat(conv1_b.astype(f32), 576)[:, None]],
        axis=1).astype(bf16)
    w2core = jnp.dot(conv2_w.astype(bf16).reshape(72, 25), jnp.asarray(_KB2),
                     preferred_element_type=f32)
    w2core = jnp.transpose(w2core.reshape(12, 6, 64, 144),
                           (0, 2, 1, 3)).reshape(768, 864)
    w2a = jnp.concatenate(
        [w2core, jnp.repeat(conv2_b.astype(f32), 64)[:, None]],
        axis=1).astype(bf16)

    w1f = jnp.concatenate([fc1_w.astype(f32), fc1_b.astype(f32)[:, None]], axis=1)
    w2f = jnp.concatenate([fc2_w.astype(f32), fc2_b.astype(f32)[:, None]], axis=1)
    w3f = jnp.concatenate([out_w.astype(f32), out_b.astype(f32)[:, None]], axis=1)

    def whole(a):
        return pl.BlockSpec(a.shape, lambda i: (0,) * a.ndim)

    out = pl.pallas_call(
        _lenet_body,
        out_shape=jax.ShapeDtypeStruct((nt, 10, _B), f32),
        grid=(nt,),
        in_specs=[
            pl.BlockSpec((_B, 784), lambda i: (i, 0)),
            whole(w1a), whole(w2a), whole(w1f), whole(w2f), whole(w3f),
        ],
        out_specs=pl.BlockSpec((1, 10, _B), lambda i: (i, 0, 0)),
        compiler_params=pltpu.CompilerParams(
            dimension_semantics=("parallel",),
            vmem_limit_bytes=48 * 1024 * 1024,
        ),
    )(x2, w1a, w2a, w1f, w2f, w3f)

    return jnp.transpose(out, (0, 2, 1)).reshape(npad, 10)[:n]
